# Initial kernel scaffold; baseline (speedup 1.0000x reference)
import jax
import jax.numpy as jnp
from jax.experimental import pallas as pl


def kernel(x, W_enc, b_enc):
    raise NotImplementedError("write your pallas kernel here")


# speedup vs baseline: 12.1391x; 12.1391x over previous
import jax
import jax.numpy as jnp
from jax.experimental import pallas as pl
from jax.experimental.pallas import tpu as pltpu

_K = 64


def _sae_block_kernel(x_ref, w_ref, b_ref, o_ref):
    zd32 = jnp.dot(x_ref[...], w_ref[...], preferred_element_type=jnp.float32)
    s = jnp.maximum(zd32 + b_ref[...].astype(jnp.float32), 0.0)

    R, C = s.shape
    sbits = jax.lax.bitcast_convert_type(s, jnp.int32)
    iota = jax.lax.broadcasted_iota(jnp.int32, (R, C), 1)
    key = (sbits | 0xFFFF) ^ iota
    v32 = jax.lax.bitcast_convert_type(key & jnp.int32(-65536), jnp.float32)
    z = v32.astype(jnp.bfloat16)

    def body(_, lohi):
        lo, hi = lohi
        mid = lo + ((hi - lo) >> 1)
        cnt = jnp.sum((key >= mid).astype(jnp.int32), axis=1, keepdims=True)
        ge = cnt >= _K
        return jnp.where(ge, mid, lo), jnp.where(ge, hi, mid)

    lo0 = jnp.zeros((R, 1), jnp.int32)
    hi0 = jnp.full((R, 1), jnp.int32(2**31 - 1))
    lo, _ = jax.lax.fori_loop(0, 31, body, (lo0, hi0))

    o_ref[...] = jnp.where(key >= lo, z, jnp.bfloat16(0))


def kernel(x, W_enc, b_enc):
    N, D = x.shape
    _, C = W_enc.shape
    R = 64
    b2 = b_enc.reshape(1, C)

    return pl.pallas_call(
        _sae_block_kernel,
        grid=(N // R,),
        in_specs=[
            pl.BlockSpec((R, D), lambda i: (i, 0)),
            pl.BlockSpec((D, C), lambda i: (0, 0)),
            pl.BlockSpec((1, C), lambda i: (0, 0)),
        ],
        out_specs=pl.BlockSpec((R, C), lambda i: (i, 0)),
        out_shape=jax.ShapeDtypeStruct((N, C), jnp.bfloat16),
    )(x, W_enc, b2)
